# baseline (speedup 1.0000x reference)
import dataclasses
import functools

import jax
import jax.numpy as jnp
from jax import lax
from jax.experimental import pallas as pl
from jax.experimental.pallas import tpu as pltpu
from jax.experimental.pallas import tpu_sc as plsc

_R = 50
_CUTOFF = 10.0


def _silu(v):
    return v * jax.nn.sigmoid(v)


def _unpack_bf16_pairs(p):
    u = lax.bitcast_convert_type(p, jnp.uint32)
    lo = lax.bitcast_convert_type(u << 16, jnp.float32)
    hi = lax.bitcast_convert_type(u & jnp.uint32(0xFFFF0000), jnp.float32)
    return jnp.concatenate([lo, hi], axis=1)


def _prep_body(x_ref, wa_ref, wb_ref, b0_ref, xa_ref, xb_ref):
    xv = x_ref[...].astype(jnp.bfloat16)
    xa_ref[...] = jnp.dot(xv, wa_ref[...], preferred_element_type=jnp.float32) + b0_ref[...]
    xb_ref[...] = jnp.dot(xv, wb_ref[...], preferred_element_type=jnp.float32)


def _prep(x, waT, wbT, b0row, bn):
    n, h = x.shape
    grid = (n // bn,)
    full = lambda shape: pl.BlockSpec(shape, lambda i: (0, 0))
    return pl.pallas_call(
        _prep_body,
        grid=grid,
        in_specs=[
            pl.BlockSpec((bn, h), lambda i: (i, 0)),
            full(waT.shape),
            full(wbT.shape),
            full(b0row.shape),
        ],
        out_specs=[
            pl.BlockSpec((bn, h), lambda i: (i, 0)),
            pl.BlockSpec((bn, h), lambda i: (i, 0)),
        ],
        out_shape=[
            jax.ShapeDtypeStruct((n, h), jnp.float32),
            jax.ShapeDtypeStruct((n, h), jnp.float32),
        ],
    )(x, waT.astype(jnp.bfloat16), wbT.astype(jnp.bfloat16), b0row)


def _sc_gather_fused(xa, xb, row1d, col1d, w, wpw, packed=True):
    n, h = xa.shape
    hw = h // 2 if packed else h
    odt = jnp.int32 if packed else jnp.float32
    e = row1d.shape[0]
    mesh = plsc.VectorSubcoreMesh(core_axis_name="core", subcore_axis_name="subcore")
    cp = pltpu.CompilerParams()
    if packed and "needs_layout_passes" in pltpu.CompilerParams.__dataclass_fields__:
        cp = dataclasses.replace(cp, needs_layout_passes=False)

    @functools.partial(
        pl.kernel,
        out_type=jax.ShapeDtypeStruct((e, hw), odt),
        mesh=mesh,
        compiler_params=cp,
        cost_estimate=pl.CostEstimate(
            flops=2 * e * h,
            bytes_accessed=e * (2 * h * 4 + hw * 4 + 8),
            transcendentals=0,
        ),
        scratch_types=[
            pltpu.VMEM((w,), jnp.int32), pltpu.VMEM((w,), jnp.int32),
            pltpu.VMEM((w,), jnp.int32), pltpu.VMEM((w,), jnp.int32),
            pltpu.VMEM((w, h), jnp.float32), pltpu.VMEM((w, h), jnp.float32),
            pltpu.VMEM((w, h), jnp.float32), pltpu.VMEM((w, h), jnp.float32),
            pltpu.VMEM((w, hw), odt), pltpu.VMEM((w, hw), odt),
            pltpu.SemaphoreType.DMA, pltpu.SemaphoreType.DMA,
            pltpu.SemaphoreType.DMA, pltpu.SemaphoreType.DMA,
            pltpu.SemaphoreType.DMA, pltpu.SemaphoreType.DMA,
            pltpu.SemaphoreType.DMA,
        ],
    )
    def k(xa_hbm, xb_hbm, row_hbm, col_hbm, g_hbm,
          r0, r1, c0, c1, a0, a1, b0, b1, o0, o1,
          sa0, sa1, sb0, sb1, so0, so1, si):
        cid = lax.axis_index("core")
        sid = lax.axis_index("subcore")
        wid = sid * 2 + cid
        base = wid * wpw
        slots = ((r0, c0, a0, b0, o0, sa0, sb0, so0),
                 (r1, c1, a1, b1, o1, sa1, sb1, so1))

        def fire(slot, widx):
            r, c, a, b, _, sa, sb, _ = slots[slot]
            pltpu.async_copy(row_hbm.at[pl.ds(widx * w, w)], r, si).wait()
            pltpu.async_copy(col_hbm.at[pl.ds(widx * w, w)], c, si).wait()
            pltpu.async_copy(xa_hbm.at[r], a, sa)
            pltpu.async_copy(xb_hbm.at[c], b, sb)

        fire(0, base)

        def process(slot, g):
            r, c, a, b, o, sa, sb, so = slots[slot]

            @pl.when(g + 1 < wpw)
            def _():
                fire(1 - slot, base + g + 1)

            pltpu.make_async_copy(xa_hbm.at[r], a, sa).wait()
            pltpu.make_async_copy(xb_hbm.at[c], b, sb).wait()

            @pl.when(g >= 2)
            def _():
                pltpu.make_async_copy(o, g_hbm.at[pl.ds(0, w)], so).wait()

            if packed:
                @pl.loop(0, w)
                def _(i):
                    for j in range(h // 32):
                        lo = (a[i, pl.ds(16 * j, 16)] + b[i, pl.ds(16 * j, 16)])
                        hi = (a[i, pl.ds(16 * j + hw, 16)]
                              + b[i, pl.ds(16 * j + hw, 16)])
                        pk = plsc.pack(lo, hi, format=plsc.PackFormat.INTERLEAVED)
                        o[i, pl.ds(16 * j, 16)] = plsc.bitcast(pk, jnp.int32)
            else:
                @pl.loop(0, w)
                def _(i):
                    for j in range(h // 16):
                        o[i, pl.ds(16 * j, 16)] = (a[i, pl.ds(16 * j, 16)]
                                                   + b[i, pl.ds(16 * j, 16)])

            pltpu.async_copy(o, g_hbm.at[pl.ds((base + g) * w, w)], so)

        @pl.loop(0, wpw)
        def _(g):
            for slot in (0, 1):
                @pl.when((g & 1) == slot)
                def _():
                    process(slot, g)

        pltpu.make_async_copy(o0, g_hbm.at[pl.ds(0, w)], so0).wait()
        pltpu.make_async_copy(o1, g_hbm.at[pl.ds(0, w)], so1).wait()

    return k(xa, xb, row1d, col1d)


def _sc_gather_ep(xa, xb, rowg, colg, window):
    n, h = xa.shape
    hw = h // 2
    nwin, w = rowg.shape
    e = nwin * w
    mesh = plsc.VectorSubcoreMesh(core_axis_name="core", subcore_axis_name="subcore")
    cp = pltpu.CompilerParams()
    if "needs_layout_passes" in pltpu.CompilerParams.__dataclass_fields__:
        cp = dataclasses.replace(cp, needs_layout_passes=False)

    @functools.partial(
        pl.kernel,
        out_type=jax.ShapeDtypeStruct((e, hw), jnp.int32),
        mesh=mesh,
        compiler_params=cp,
        scratch_types=[
            pltpu.VMEM((w, h), jnp.float32),
            pltpu.VMEM((w, h), jnp.float32),
        ],
    )
    def k(xa_hbm, xb_hbm, row_hbm, col_hbm, g_hbm, bufa, bufb):
        def body(r_vmem, c_vmem, o_vmem):
            pltpu.sync_copy(xa_hbm.at[r_vmem.at[0]], bufa)
            pltpu.sync_copy(xb_hbm.at[c_vmem.at[0]], bufb)

            @pl.loop(0, w)
            def _(i):
                for j in range(h // 32):
                    lo = (bufa[i, pl.ds(16 * j, 16)] + bufb[i, pl.ds(16 * j, 16)])
                    hi = (bufa[i, pl.ds(16 * j + hw, 16)]
                          + bufb[i, pl.ds(16 * j + hw, 16)])
                    pk = plsc.pack(lo, hi, format=plsc.PackFormat.INTERLEAVED)
                    o_vmem.at[i, pl.ds(16 * j, 16)][...] = plsc.bitcast(pk, jnp.int32)

        pltpu.emit_pipeline(
            body,
            grid=(nwin,),
            in_specs=[
                pl.BlockSpec((1, w), lambda i: (i, 0)),
                pl.BlockSpec((1, w), lambda i: (i, 0)),
            ],
            out_specs=[pl.BlockSpec((w, hw), lambda i: (i, 0))],
            core_axis_name=("core", "subcore"),
            dimension_semantics=(pltpu.PARALLEL,),
        )(row_hbm, col_hbm, g_hbm)

    return k(xa, xb, rowg, colg)


def _sc_gather(xa, xb, rowg, colg, window):
    n, h = xa.shape
    e = rowg.shape[0] * rowg.shape[1]
    mesh = plsc.VectorSubcoreMesh(core_axis_name="core", subcore_axis_name="subcore")
    out_ty = (
        jax.ShapeDtypeStruct((e, h), xa.dtype),
        jax.ShapeDtypeStruct((e, h), xa.dtype),
    )

    @functools.partial(pl.kernel, out_type=out_ty, mesh=mesh)
    def k(xa_hbm, xb_hbm, row_hbm, col_hbm, ga_hbm, gb_hbm):
        def body(r_vmem, c_vmem, ga_vmem, gb_vmem):
            pltpu.sync_copy(xa_hbm.at[r_vmem.at[0]], ga_vmem)
            pltpu.sync_copy(xb_hbm.at[c_vmem.at[0]], gb_vmem)

        pltpu.emit_pipeline(
            body,
            grid=(e // window,),
            in_specs=[
                pl.BlockSpec((1, window), lambda i: (i, 0)),
                pl.BlockSpec((1, window), lambda i: (i, 0)),
            ],
            out_specs=[
                pl.BlockSpec((window, h), lambda i: (i, 0)),
                pl.BlockSpec((window, h), lambda i: (i, 0)),
            ],
            core_axis_name=("core", "subcore"),
            dimension_semantics=(pltpu.PARALLEL,),
        )(row_hbm, col_hbm, ga_hbm, gb_hbm)

    return k(xa, xb, rowg, colg)


def _edge_body(g_ref, d_ref, w0cT_ref, w1T_ref, b1_ref, w2T_ref,
               b2_ref, m_ref):
    d = d_ref[...]
    step = _CUTOFF / (_R - 1)
    coeff = -0.5 / (step * step)
    offs = lax.broadcasted_iota(jnp.int32, (1, _R), 1).astype(jnp.float32) * step
    rf = jnp.exp(coeff * (d - offs) ** 2)
    gv = g_ref[...]
    gx = _unpack_bf16_pairs(gv) if gv.dtype == jnp.int32 else gv
    pre = (gx
           + jnp.dot(rf.astype(jnp.bfloat16), w0cT_ref[...],
                     preferred_element_type=jnp.float32))
    m0 = _silu(pre)
    m1 = _silu(jnp.dot(m0.astype(jnp.bfloat16), w1T_ref[...],
                       preferred_element_type=jnp.float32) + b1_ref[...])
    m_ref[...] = jnp.dot(m1.astype(jnp.bfloat16), w2T_ref[...],
                         preferred_element_type=jnp.float32) + b2_ref[...]


def _edge_mlp(g, dcol, w0cT, w1T, b1row, w2T, b2row, be):
    e, hp = g.shape
    h = 2 * hp if g.dtype == jnp.int32 else hp
    grid = (e // be,)
    full = lambda a: pl.BlockSpec(a.shape, lambda i: tuple(0 for _ in a.shape))
    return pl.pallas_call(
        _edge_body,
        grid=grid,
        in_specs=[
            pl.BlockSpec((be, hp), lambda i: (i, 0)),
            pl.BlockSpec((be, 1), lambda i: (i, 0)),
            full(w0cT),
            full(w1T),
            full(b1row),
            full(w2T),
            full(b2row),
        ],
        out_specs=pl.BlockSpec((be, h), lambda i: (i, 0)),
        out_shape=jax.ShapeDtypeStruct((e, h), jnp.float32),
    )(g, dcol, w0cT, w1T, b1row, w2T, b2row)


def _sc_scatter(ms, colgs, n, window):
    nchunk = len(ms)
    e, h = ms[0].shape
    nsub = 16
    npad = (n + 16 * 8 * 5 - 1) // (16 * 8 * 5) * (16 * 8 * 5)
    nps = npad // nsub
    zr = nps // 10
    mesh = plsc.VectorSubcoreMesh(core_axis_name="core", subcore_axis_name="subcore")

    @functools.partial(
        pl.kernel,
        out_type=jax.ShapeDtypeStruct((2, npad, h), jnp.float32),
        mesh=mesh,
        scratch_types=[
            pltpu.VMEM_SHARED((npad, h), jnp.float32),
            pltpu.VMEM((zr, h), jnp.float32),
        ],
    )
    def k(*refs):
        m_hbms = refs[:nchunk]
        col_hbms = refs[nchunk:2 * nchunk]
        out_hbm = refs[2 * nchunk]
        acc = refs[2 * nchunk + 1]
        zbuf = refs[2 * nchunk + 2]
        cid = lax.axis_index("core")
        sid = lax.axis_index("subcore")

        @pl.loop(0, zr)
        def _(i):
            @pl.loop(0, h, step=16)
            def _(j):
                zbuf.at[i, pl.ds(j, 16)][...] = jnp.zeros((16,), jnp.float32)

        @pl.loop(0, nps, step=zr)
        def _(r):
            pltpu.sync_copy(zbuf, acc.at[pl.ds(sid * nps + r, zr)])

        plsc.subcore_barrier()

        def body(m_vmem, i_vmem):
            pltpu.sync_copy(m_vmem, acc.at[i_vmem.at[0]], add=True)

        for j in range(nchunk):
            ej = m_hbms[j].shape[0]
            pltpu.emit_pipeline(
                body,
                grid=(ej // window,),
                in_specs=[
                    pl.BlockSpec((window, h), lambda i: (i, 0)),
                    pl.BlockSpec((1, window), lambda i: (i, 0)),
                ],
                out_specs=[],
                core_axis_name=("core", "subcore"),
                dimension_semantics=(pltpu.PARALLEL,),
            )(m_hbms[j], col_hbms[j])

        plsc.subcore_barrier()

        @pl.loop(0, nps, step=zr)
        def _(r):
            pltpu.sync_copy(acc.at[pl.ds(sid * nps + r, zr)],
                            out_hbm.at[cid, pl.ds(sid * nps + r, zr)])

    return k(*ms, *colgs)


def _node_body(x_ref, a0_ref, a1_ref, uwaT_ref, uwbT_ref, ub0_ref, uw1T_ref,
               ub1_ref, iw0T_ref, ib0_ref, iw1T_ref, ib1_ref, v_ref):
    bf = jnp.bfloat16
    agg = (a0_ref[0] + a1_ref[0]).astype(bf)
    u = _silu(jnp.dot(x_ref[...].astype(bf), uwaT_ref[...], preferred_element_type=jnp.float32)
              + jnp.dot(agg, uwbT_ref[...], preferred_element_type=jnp.float32)
              + ub0_ref[...])
    u = jnp.dot(u.astype(bf), uw1T_ref[...], preferred_element_type=jnp.float32) + ub1_ref[...]
    v = _silu(jnp.dot(u.astype(bf), iw0T_ref[...], preferred_element_type=jnp.float32) + ib0_ref[...])
    v_ref[...] = jnp.dot(v.astype(bf), iw1T_ref[...], preferred_element_type=jnp.float32) + ib1_ref[...]


def _node_mlp(x, agg2, uwaT, uwbT, ub0row, uw1T, ub1row, iw0T, ib0row, iw1T,
              ib1row, bn):
    n, h = x.shape
    grid = (n // bn,)
    full = lambda a: pl.BlockSpec(a.shape, lambda i: tuple(0 for _ in a.shape))
    return pl.pallas_call(
        _node_body,
        grid=grid,
        in_specs=[
            pl.BlockSpec((bn, h), lambda i: (i, 0)),
            pl.BlockSpec((1, bn, h), lambda i: (0, i, 0)),
            pl.BlockSpec((1, bn, h), lambda i: (1, i, 0)),
            full(uwaT), full(uwbT), full(ub0row), full(uw1T), full(ub1row),
            full(iw0T), full(ib0row), full(iw1T), full(ib1row),
        ],
        out_specs=pl.BlockSpec((bn, h), lambda i: (i, 0)),
        out_shape=jax.ShapeDtypeStruct((n, h), jnp.float32),
    )(x, agg2, agg2, uwaT, uwbT, ub0row, uw1T, ub1row, iw0T, ib0row, iw1T, ib1row)


def kernel(x, edge_index, edge_attr, distances, msg_W0, msg_b0, msg_W1,
           msg_b1, msg_W2, msg_b2, upd_W0, upd_b0, upd_W1, upd_b1, int_W0,
           int_b0, int_W1, int_b1):
    n, h = x.shape
    e = edge_index.shape[1]

    waT = msg_W0[:, :h].T
    wbT = msg_W0[:, h:2 * h].T
    w0cT = msg_W0[:, 2 * h:].T
    b0row = msg_b0.reshape(1, -1)
    w1T, b1row = msg_W1.T, msg_b1.reshape(1, -1)
    w2T, b2row = msg_W2.T, msg_b2.reshape(1, -1)
    uwaT = upd_W0[:, :h].T
    uwbT = upd_W0[:, h:].T
    ub0row = upd_b0.reshape(1, -1)
    uw1T, ub1row = upd_W1.T, upd_b1.reshape(1, -1)
    iw0T, ib0row = int_W0.T, int_b0.reshape(1, -1)
    iw1T, ib1row = int_W1.T, int_b1.reshape(1, -1)

    row2d = edge_index[0].reshape(1, e).astype(jnp.int32)
    col2d = edge_index[1].reshape(1, e).astype(jnp.int32)
    dcol = distances.reshape(e, 1)

    bf = jnp.bfloat16
    xa, xb = _prep(x, waT, wbT, b0row, bn=2000)

    gw = 80
    sw = 128
    chunk_sizes = (163840, 156160)
    use_packed = "ep"
    row1 = row2d[0]
    col1 = col2d[0]
    ms, cols = [], []
    off = 0
    for ec in chunk_sizes:
        sl = slice(off, off + ec)
        wpw = ec // (32 * gw)
        if use_packed == "ep":
            g = _sc_gather_ep(xa, xb, row1[sl].reshape(ec // gw, gw),
                              col1[sl].reshape(ec // gw, gw), gw)
        else:
            g = _sc_gather_fused(xa, xb, row1[sl], col1[sl], gw, wpw,
                                 packed=use_packed)
        mk = _edge_mlp(g, dcol[sl], w0cT.astype(bf), w1T.astype(bf),
                       b1row, w2T.astype(bf), b2row, be=1280)
        ms.append(mk)
        cols.append(col1[sl].reshape(ec // sw, sw))
        off += ec
    agg2 = _sc_scatter(ms, cols, n, window=sw)
    return _node_mlp(x, agg2, uwaT.astype(bf), uwbT.astype(bf), ub0row,
                     uw1T.astype(bf), ub1row, iw0T.astype(bf), ib0row,
                     iw1T.astype(bf), ib1row, bn=2000)


# speedup vs baseline: 1.3286x; 1.3286x over previous
import dataclasses
import functools

import jax
import jax.numpy as jnp
from jax import lax
from jax.experimental import pallas as pl
from jax.experimental.pallas import tpu as pltpu
from jax.experimental.pallas import tpu_sc as plsc

_R = 50
_CUTOFF = 10.0


def _silu(v):
    return v * jax.nn.sigmoid(v)


def _unpack_bf16_pairs(p):
    u = lax.bitcast_convert_type(p, jnp.uint32)
    lo = lax.bitcast_convert_type(u << 16, jnp.float32)
    hi = lax.bitcast_convert_type(u & jnp.uint32(0xFFFF0000), jnp.float32)
    return jnp.concatenate([lo, hi], axis=1)


def _prep_body(x_ref, wa_ref, wb_ref, b0_ref, xa_ref, xb_ref):
    xv = x_ref[...].astype(jnp.bfloat16)
    xa_ref[...] = jnp.dot(xv, wa_ref[...], preferred_element_type=jnp.float32) + b0_ref[...]
    xb_ref[...] = jnp.dot(xv, wb_ref[...], preferred_element_type=jnp.float32)


def _prep(x, waT, wbT, b0row, bn):
    n, h = x.shape
    grid = (n // bn,)
    full = lambda shape: pl.BlockSpec(shape, lambda i: (0, 0))
    return pl.pallas_call(
        _prep_body,
        grid=grid,
        in_specs=[
            pl.BlockSpec((bn, h), lambda i: (i, 0)),
            full(waT.shape),
            full(wbT.shape),
            full(b0row.shape),
        ],
        out_specs=[
            pl.BlockSpec((bn, h), lambda i: (i, 0)),
            pl.BlockSpec((bn, h), lambda i: (i, 0)),
        ],
        out_shape=[
            jax.ShapeDtypeStruct((n, h), jnp.float32),
            jax.ShapeDtypeStruct((n, h), jnp.float32),
        ],
    )(x, waT.astype(jnp.bfloat16), wbT.astype(jnp.bfloat16), b0row)


def _sc_gather_fused(xa, xb, row1d, col1d, w, wpw, packed=True):
    n, h = xa.shape
    hw = h // 2 if packed else h
    odt = jnp.int32 if packed else jnp.float32
    e = row1d.shape[0]
    mesh = plsc.VectorSubcoreMesh(core_axis_name="core", subcore_axis_name="subcore")
    cp = pltpu.CompilerParams()
    if packed and "needs_layout_passes" in pltpu.CompilerParams.__dataclass_fields__:
        cp = dataclasses.replace(cp, needs_layout_passes=False)

    @functools.partial(
        pl.kernel,
        out_type=jax.ShapeDtypeStruct((e, hw), odt),
        mesh=mesh,
        compiler_params=cp,
        cost_estimate=pl.CostEstimate(
            flops=2 * e * h,
            bytes_accessed=e * (2 * h * 4 + hw * 4 + 8),
            transcendentals=0,
        ),
        scratch_types=[
            pltpu.VMEM((w,), jnp.int32), pltpu.VMEM((w,), jnp.int32),
            pltpu.VMEM((w,), jnp.int32), pltpu.VMEM((w,), jnp.int32),
            pltpu.VMEM((w, h), jnp.float32), pltpu.VMEM((w, h), jnp.float32),
            pltpu.VMEM((w, h), jnp.float32), pltpu.VMEM((w, h), jnp.float32),
            pltpu.VMEM((w, hw), odt), pltpu.VMEM((w, hw), odt),
            pltpu.SemaphoreType.DMA, pltpu.SemaphoreType.DMA,
            pltpu.SemaphoreType.DMA, pltpu.SemaphoreType.DMA,
            pltpu.SemaphoreType.DMA, pltpu.SemaphoreType.DMA,
            pltpu.SemaphoreType.DMA,
        ],
    )
    def k(xa_hbm, xb_hbm, row_hbm, col_hbm, g_hbm,
          r0, r1, c0, c1, a0, a1, b0, b1, o0, o1,
          sa0, sa1, sb0, sb1, so0, so1, si):
        cid = lax.axis_index("core")
        sid = lax.axis_index("subcore")
        wid = sid * 2 + cid
        base = wid * wpw
        slots = ((r0, c0, a0, b0, o0, sa0, sb0, so0),
                 (r1, c1, a1, b1, o1, sa1, sb1, so1))

        def fire(slot, widx):
            r, c, a, b, _, sa, sb, _ = slots[slot]
            pltpu.async_copy(row_hbm.at[pl.ds(widx * w, w)], r, si).wait()
            pltpu.async_copy(col_hbm.at[pl.ds(widx * w, w)], c, si).wait()
            pltpu.async_copy(xa_hbm.at[r], a, sa)
            pltpu.async_copy(xb_hbm.at[c], b, sb)

        fire(0, base)

        def process(slot, g):
            r, c, a, b, o, sa, sb, so = slots[slot]

            @pl.when(g + 1 < wpw)
            def _():
                fire(1 - slot, base + g + 1)

            pltpu.make_async_copy(xa_hbm.at[r], a, sa).wait()
            pltpu.make_async_copy(xb_hbm.at[c], b, sb).wait()

            @pl.when(g >= 2)
            def _():
                pltpu.make_async_copy(o, g_hbm.at[pl.ds(0, w)], so).wait()

            if packed:
                @pl.loop(0, w)
                def _(i):
                    for j in range(h // 32):
                        lo = (a[i, pl.ds(16 * j, 16)] + b[i, pl.ds(16 * j, 16)])
                        hi = (a[i, pl.ds(16 * j + hw, 16)]
                              + b[i, pl.ds(16 * j + hw, 16)])
                        pk = plsc.pack(lo, hi, format=plsc.PackFormat.INTERLEAVED)
                        o[i, pl.ds(16 * j, 16)] = plsc.bitcast(pk, jnp.int32)
            else:
                @pl.loop(0, w)
                def _(i):
                    for j in range(h // 16):
                        o[i, pl.ds(16 * j, 16)] = (a[i, pl.ds(16 * j, 16)]
                                                   + b[i, pl.ds(16 * j, 16)])

            pltpu.async_copy(o, g_hbm.at[pl.ds((base + g) * w, w)], so)

        @pl.loop(0, wpw)
        def _(g):
            for slot in (0, 1):
                @pl.when((g & 1) == slot)
                def _():
                    process(slot, g)

        pltpu.make_async_copy(o0, g_hbm.at[pl.ds(0, w)], so0).wait()
        pltpu.make_async_copy(o1, g_hbm.at[pl.ds(0, w)], so1).wait()

    return k(xa, xb, row1d, col1d)


def _sc_gather_ep(xa, xb, rowg, colg, window):
    n, h = xa.shape
    hw = h // 2
    nwin, w = rowg.shape
    e = nwin * w
    mesh = plsc.VectorSubcoreMesh(core_axis_name="core", subcore_axis_name="subcore")
    cp = pltpu.CompilerParams()
    if "needs_layout_passes" in pltpu.CompilerParams.__dataclass_fields__:
        cp = dataclasses.replace(cp, needs_layout_passes=False)

    @functools.partial(
        pl.kernel,
        out_type=jax.ShapeDtypeStruct((e, hw), jnp.int32),
        mesh=mesh,
        compiler_params=cp,
        scratch_types=[
            pltpu.VMEM((w, h), jnp.float32),
            pltpu.VMEM((w, h), jnp.float32),
        ],
    )
    def k(xa_hbm, xb_hbm, row_hbm, col_hbm, g_hbm, bufa, bufb):
        def body(r_vmem, c_vmem, o_vmem):
            pltpu.sync_copy(xa_hbm.at[r_vmem.at[0]], bufa)
            pltpu.sync_copy(xb_hbm.at[c_vmem.at[0]], bufb)

            @pl.loop(0, w)
            def _(i):
                for j in range(h // 32):
                    lo = (bufa[i, pl.ds(16 * j, 16)] + bufb[i, pl.ds(16 * j, 16)])
                    hi = (bufa[i, pl.ds(16 * j + hw, 16)]
                          + bufb[i, pl.ds(16 * j + hw, 16)])
                    pk = plsc.pack(lo, hi, format=plsc.PackFormat.INTERLEAVED)
                    o_vmem.at[i, pl.ds(16 * j, 16)][...] = plsc.bitcast(pk, jnp.int32)

        pltpu.emit_pipeline(
            body,
            grid=(nwin,),
            in_specs=[
                pl.BlockSpec((1, w), lambda i: (i, 0)),
                pl.BlockSpec((1, w), lambda i: (i, 0)),
            ],
            out_specs=[pl.BlockSpec((w, hw), lambda i: (i, 0))],
            core_axis_name=("core", "subcore"),
            dimension_semantics=(pltpu.PARALLEL,),
        )(row_hbm, col_hbm, g_hbm)

    return k(xa, xb, rowg, colg)


def _sc_gather(xa, xb, rowg, colg, window):
    n, h = xa.shape
    e = rowg.shape[0] * rowg.shape[1]
    mesh = plsc.VectorSubcoreMesh(core_axis_name="core", subcore_axis_name="subcore")
    out_ty = (
        jax.ShapeDtypeStruct((e, h), xa.dtype),
        jax.ShapeDtypeStruct((e, h), xa.dtype),
    )

    @functools.partial(pl.kernel, out_type=out_ty, mesh=mesh)
    def k(xa_hbm, xb_hbm, row_hbm, col_hbm, ga_hbm, gb_hbm):
        def body(r_vmem, c_vmem, ga_vmem, gb_vmem):
            pltpu.sync_copy(xa_hbm.at[r_vmem.at[0]], ga_vmem)
            pltpu.sync_copy(xb_hbm.at[c_vmem.at[0]], gb_vmem)

        pltpu.emit_pipeline(
            body,
            grid=(e // window,),
            in_specs=[
                pl.BlockSpec((1, window), lambda i: (i, 0)),
                pl.BlockSpec((1, window), lambda i: (i, 0)),
            ],
            out_specs=[
                pl.BlockSpec((window, h), lambda i: (i, 0)),
                pl.BlockSpec((window, h), lambda i: (i, 0)),
            ],
            core_axis_name=("core", "subcore"),
            dimension_semantics=(pltpu.PARALLEL,),
        )(row_hbm, col_hbm, ga_hbm, gb_hbm)

    return k(xa, xb, rowg, colg)


def _edge_body(g_ref, d_ref, w0cT_ref, w1T_ref, b1_ref, w2T_ref,
               b2_ref, m_ref):
    d = d_ref[...]
    step = _CUTOFF / (_R - 1)
    coeff = -0.5 / (step * step)
    offs = lax.broadcasted_iota(jnp.int32, (1, _R), 1).astype(jnp.float32) * step
    rf = jnp.exp(coeff * (d - offs) ** 2)
    gv = g_ref[...]
    gx = _unpack_bf16_pairs(gv) if gv.dtype == jnp.int32 else gv
    pre = (gx
           + jnp.dot(rf.astype(jnp.bfloat16), w0cT_ref[...],
                     preferred_element_type=jnp.float32))
    m0 = _silu(pre)
    m1 = _silu(jnp.dot(m0.astype(jnp.bfloat16), w1T_ref[...],
                       preferred_element_type=jnp.float32) + b1_ref[...])
    m_ref[...] = jnp.dot(m1.astype(jnp.bfloat16), w2T_ref[...],
                         preferred_element_type=jnp.float32) + b2_ref[...]


def _edge_mlp(g, dcol, w0cT, w1T, b1row, w2T, b2row, be):
    e, hp = g.shape
    h = 2 * hp if g.dtype == jnp.int32 else hp
    grid = (e // be,)
    full = lambda a: pl.BlockSpec(a.shape, lambda i: tuple(0 for _ in a.shape))
    return pl.pallas_call(
        _edge_body,
        grid=grid,
        in_specs=[
            pl.BlockSpec((be, hp), lambda i: (i, 0)),
            pl.BlockSpec((be, 1), lambda i: (i, 0)),
            full(w0cT),
            full(w1T),
            full(b1row),
            full(w2T),
            full(b2row),
        ],
        out_specs=pl.BlockSpec((be, h), lambda i: (i, 0)),
        out_shape=jax.ShapeDtypeStruct((e, h), jnp.float32),
    )(g, dcol, w0cT, w1T, b1row, w2T, b2row)


def _sc_scatter(ms, colgs, n, window):
    nchunk = len(ms)
    e, h = ms[0].shape
    nsub = 16
    npad = (n + 16 * 8 * 5 - 1) // (16 * 8 * 5) * (16 * 8 * 5)
    nps = npad // nsub
    zr = nps // 10
    mesh = plsc.VectorSubcoreMesh(core_axis_name="core", subcore_axis_name="subcore")

    @functools.partial(
        pl.kernel,
        out_type=jax.ShapeDtypeStruct((2, npad, h), jnp.float32),
        mesh=mesh,
        scratch_types=[
            pltpu.VMEM_SHARED((npad, h), jnp.float32),
            pltpu.VMEM((zr, h), jnp.float32),
        ],
    )
    def k(*refs):
        m_hbms = refs[:nchunk]
        col_hbms = refs[nchunk:2 * nchunk]
        out_hbm = refs[2 * nchunk]
        acc = refs[2 * nchunk + 1]
        zbuf = refs[2 * nchunk + 2]
        cid = lax.axis_index("core")
        sid = lax.axis_index("subcore")

        @pl.loop(0, zr)
        def _(i):
            @pl.loop(0, h, step=16)
            def _(j):
                zbuf.at[i, pl.ds(j, 16)][...] = jnp.zeros((16,), jnp.float32)

        @pl.loop(0, nps, step=zr)
        def _(r):
            pltpu.sync_copy(zbuf, acc.at[pl.ds(sid * nps + r, zr)])

        plsc.subcore_barrier()

        def body(m_vmem, i_vmem):
            pltpu.sync_copy(m_vmem, acc.at[i_vmem.at[0]], add=True)

        for j in range(nchunk):
            ej = m_hbms[j].shape[0]
            pltpu.emit_pipeline(
                body,
                grid=(ej // window,),
                in_specs=[
                    pl.BlockSpec((window, h), lambda i: (i, 0)),
                    pl.BlockSpec((1, window), lambda i: (i, 0)),
                ],
                out_specs=[],
                core_axis_name=("core", "subcore"),
                dimension_semantics=(pltpu.PARALLEL,),
            )(m_hbms[j], col_hbms[j])

        plsc.subcore_barrier()

        @pl.loop(0, nps, step=zr)
        def _(r):
            pltpu.sync_copy(acc.at[pl.ds(sid * nps + r, zr)],
                            out_hbm.at[cid, pl.ds(sid * nps + r, zr)])

    return k(*ms, *colgs)


def _node_body(x_ref, a0_ref, a1_ref, uwaT_ref, uwbT_ref, ub0_ref, uw1T_ref,
               ub1_ref, iw0T_ref, ib0_ref, iw1T_ref, ib1_ref, v_ref):
    bf = jnp.bfloat16
    agg = (a0_ref[0] + a1_ref[0]).astype(bf)
    u = _silu(jnp.dot(x_ref[...].astype(bf), uwaT_ref[...], preferred_element_type=jnp.float32)
              + jnp.dot(agg, uwbT_ref[...], preferred_element_type=jnp.float32)
              + ub0_ref[...])
    u = jnp.dot(u.astype(bf), uw1T_ref[...], preferred_element_type=jnp.float32) + ub1_ref[...]
    v = _silu(jnp.dot(u.astype(bf), iw0T_ref[...], preferred_element_type=jnp.float32) + ib0_ref[...])
    v_ref[...] = jnp.dot(v.astype(bf), iw1T_ref[...], preferred_element_type=jnp.float32) + ib1_ref[...]


def _node_mlp(x, agg2, uwaT, uwbT, ub0row, uw1T, ub1row, iw0T, ib0row, iw1T,
              ib1row, bn):
    n, h = x.shape
    grid = (n // bn,)
    full = lambda a: pl.BlockSpec(a.shape, lambda i: tuple(0 for _ in a.shape))
    return pl.pallas_call(
        _node_body,
        grid=grid,
        in_specs=[
            pl.BlockSpec((bn, h), lambda i: (i, 0)),
            pl.BlockSpec((1, bn, h), lambda i: (0, i, 0)),
            pl.BlockSpec((1, bn, h), lambda i: (1, i, 0)),
            full(uwaT), full(uwbT), full(ub0row), full(uw1T), full(ub1row),
            full(iw0T), full(ib0row), full(iw1T), full(ib1row),
        ],
        out_specs=pl.BlockSpec((bn, h), lambda i: (i, 0)),
        out_shape=jax.ShapeDtypeStruct((n, h), jnp.float32),
    )(x, agg2, agg2, uwaT, uwbT, ub0row, uw1T, ub1row, iw0T, ib0row, iw1T, ib1row)


def kernel(x, edge_index, edge_attr, distances, msg_W0, msg_b0, msg_W1,
           msg_b1, msg_W2, msg_b2, upd_W0, upd_b0, upd_W1, upd_b1, int_W0,
           int_b0, int_W1, int_b1):
    n, h = x.shape
    e = edge_index.shape[1]

    waT = msg_W0[:, :h].T
    wbT = msg_W0[:, h:2 * h].T
    w0cT = msg_W0[:, 2 * h:].T
    b0row = msg_b0.reshape(1, -1)
    w1T, b1row = msg_W1.T, msg_b1.reshape(1, -1)
    w2T, b2row = msg_W2.T, msg_b2.reshape(1, -1)
    uwaT = upd_W0[:, :h].T
    uwbT = upd_W0[:, h:].T
    ub0row = upd_b0.reshape(1, -1)
    uw1T, ub1row = upd_W1.T, upd_b1.reshape(1, -1)
    iw0T, ib0row = int_W0.T, int_b0.reshape(1, -1)
    iw1T, ib1row = int_W1.T, int_b1.reshape(1, -1)

    row2d = edge_index[0].reshape(1, e).astype(jnp.int32)
    col2d = edge_index[1].reshape(1, e).astype(jnp.int32)
    dcol = distances.reshape(e, 1)

    bf = jnp.bfloat16
    xa, xb = _prep(x, waT, wbT, b0row, bn=2000)

    gw = 80
    sw = 128
    chunk_sizes = (320000,)
    use_packed = False
    row1 = row2d[0]
    col1 = col2d[0]
    ms, cols = [], []
    off = 0
    for ec in chunk_sizes:
        sl = slice(off, off + ec)
        wpw = ec // (32 * gw)
        if use_packed == "ep":
            g = _sc_gather_ep(xa, xb, row1[sl].reshape(ec // gw, gw),
                              col1[sl].reshape(ec // gw, gw), gw)
        else:
            g = _sc_gather_fused(xa, xb, row1[sl], col1[sl], gw, wpw,
                                 packed=use_packed)
        mk = _edge_mlp(g, dcol[sl], w0cT.astype(bf), w1T.astype(bf),
                       b1row, w2T.astype(bf), b2row, be=1280)
        ms.append(mk)
        cols.append(col1[sl].reshape(ec // sw, sw))
        off += ec
    agg2 = _sc_scatter(ms, cols, n, window=sw)
    return _node_mlp(x, agg2, uwaT.astype(bf), uwbT.astype(bf), ub0row,
                     uw1T.astype(bf), ub1row, iw0T.astype(bf), ib0row,
                     iw1T.astype(bf), ib1row, bn=2000)
